# baseline (speedup 1.0000x reference)
import functools

import jax
import jax.numpy as jnp
from jax import lax
from jax.experimental import pallas as pl
from jax.experimental.pallas import tpu as pltpu
from jax.experimental.pallas import tpu_sc as plsc

_B = 32
_M = 30
_SHAPES = [(64, 64, 64), (128, 32, 32), (256, 16, 16)]
_DS = [c * h * w for (c, h, w) in _SHAPES]
_NCHUNK = 8


def _partial_dist(a, b, d):
    cross = jax.lax.dot_general(
        a, b, (((1,), (1,)), ((), ())), preferred_element_type=jnp.float32
    )
    a2 = jnp.sum(a * a, axis=1)
    b2 = jnp.sum(b * b, axis=1)
    return (a2[:, None] + b2[None, :] - 2.0 * cross) * (1.0 / d)


def _dist1_kernel(f1, m1, out, acc):
    g = pl.program_id(0)

    @pl.when(g == 0)
    def _init():
        acc[:] = jnp.zeros_like(acc)

    acc[:] += _partial_dist(f1[:], m1[:], _DS[0])

    @pl.when(g == _NCHUNK - 1)
    def _fin():
        out[:] = acc[:]


def _dist23_kernel(d1, f2, m2, f3, m3, out_dist, acc):
    g = pl.program_id(0)

    @pl.when(g == 0)
    def _init():
        acc[:] = jnp.zeros_like(acc)

    acc[:] += _partial_dist(f2[:], m2[:], _DS[1])
    acc[:] += _partial_dist(f3[:], m3[:], _DS[2])

    @pl.when(g == _NCHUNK - 1)
    def _fin():
        out_dist[:, :] = jnp.full((_B, 128), jnp.inf, dtype=jnp.float32)
        out_dist[:_M, :_B] = jnp.transpose(acc[:] + d1[:], (1, 0))


def _compute_idx(ff1, mf1, ff2, mf2, ff3, mf3):
    c1 = _DS[0] // _NCHUNK
    d1 = pl.pallas_call(
        _dist1_kernel,
        grid=(_NCHUNK,),
        in_specs=[
            pl.BlockSpec((_B, c1), lambda i: (0, i)),
            pl.BlockSpec((_M, c1), lambda i: (0, i)),
        ],
        out_specs=pl.BlockSpec((_B, _M), lambda i: (0, 0)),
        out_shape=jax.ShapeDtypeStruct((_B, _M), jnp.float32),
        scratch_shapes=[pltpu.VMEM((_B, _M), jnp.float32)],
        compiler_params=pltpu.CompilerParams(
            dimension_semantics=("arbitrary",)
        ),
    )(ff1, mf1)

    c2 = _DS[1] // _NCHUNK
    c3 = _DS[2] // _NCHUNK
    dist = pl.pallas_call(
        _dist23_kernel,
        grid=(_NCHUNK,),
        in_specs=[
            pl.BlockSpec((_B, _M), lambda i: (0, 0)),
            pl.BlockSpec((_B, c2), lambda i: (0, i)),
            pl.BlockSpec((_M, c2), lambda i: (0, i)),
            pl.BlockSpec((_B, c3), lambda i: (0, i)),
            pl.BlockSpec((_M, c3), lambda i: (0, i)),
        ],
        out_specs=pl.BlockSpec((_B, 128), lambda i: (0, 0)),
        out_shape=jax.ShapeDtypeStruct((_B, 128), jnp.float32),
        scratch_shapes=[pltpu.VMEM((_B, _M), jnp.float32)],
        compiler_params=pltpu.CompilerParams(
            dimension_semantics=("arbitrary",)
        ),
    )(d1, ff2, mf2, ff3, mf3)
    return _sc_argmin(dist)


def _sc_argmin_body(dist_hbm, idx_hbm, dist_v, idx_v):
    cid = lax.axis_index("c")
    sid = lax.axis_index("s")

    @pl.when(jnp.logical_and(cid == 0, sid == 0))
    def _():
        pltpu.sync_copy(dist_hbm, dist_v)
        for half in range(2):
            best_v = jnp.full((16,), jnp.inf, jnp.float32)
            best_i = jnp.zeros((16,), jnp.int32)
            for mrow in range(_M):
                v = dist_v[mrow, pl.ds(half * 16, 16)]
                upd = v < best_v
                best_v = jnp.where(upd, v, best_v)
                best_i = jnp.where(upd, mrow, best_i)
            idx_v[pl.ds(half * 16, 16)] = best_i
        pltpu.sync_copy(idx_v, idx_hbm)


def _sc_argmin(dist):
    mesh = plsc.VectorSubcoreMesh(core_axis_name="c", subcore_axis_name="s")
    return pl.kernel(
        _sc_argmin_body,
        out_type=jax.ShapeDtypeStruct((_B,), jnp.int32),
        mesh=mesh,
        scratch_types=[
            pltpu.VMEM((_B, 128), jnp.float32),
            pltpu.VMEM((_B,), jnp.int32),
        ],
    )(dist)


def _gather_kernel(idx_ref, f1, m1, f2, m2, f3, m3, o1, o2, o3):
    del idx_ref
    for f, m, o, (c, _, _) in (
        (f1, m1, o1, _SHAPES[0]),
        (f2, m2, o2, _SHAPES[1]),
        (f3, m3, o3, _SHAPES[2]),
    ):
        fv = f[0]
        mv = m[0]
        o[0, :, :c] = fv
        d = mv - fv
        o[0, :, c:] = d * d


def _compute_outputs(idx, fn1, mn1, fn2, mn2, fn3, mn3):
    in_specs = []
    out_specs = []
    out_shape = []
    for c, h, w in _SHAPES:
        in_specs.append(
            pl.BlockSpec((1, h * w, c), lambda b, idx_ref: (b, 0, 0))
        )
        in_specs.append(
            pl.BlockSpec(
                (1, h * w, c), lambda b, idx_ref: (idx_ref[b], 0, 0)
            )
        )
        out_specs.append(
            pl.BlockSpec((1, h * w, 2 * c), lambda b, idx_ref: (b, 0, 0))
        )
        out_shape.append(
            jax.ShapeDtypeStruct((_B, h * w, 2 * c), jnp.float32)
        )
    grid_spec = pltpu.PrefetchScalarGridSpec(
        num_scalar_prefetch=1,
        grid=(_B,),
        in_specs=in_specs,
        out_specs=out_specs,
    )
    ons = pl.pallas_call(
        _gather_kernel,
        grid_spec=grid_spec,
        out_shape=out_shape,
        compiler_params=pltpu.CompilerParams(
            dimension_semantics=("arbitrary",)
        ),
    )(idx, fn1, mn1, fn2, mn2, fn3, mn3)
    outs = []
    for on, (c, h, w) in zip(ons, _SHAPES):
        outs.append(jnp.transpose(on, (0, 2, 1)).reshape(_B, 2 * c, h, w))
    return outs


def _nhwc(x, c, h, w):
    return jnp.transpose(x, (0, 2, 3, 1)).reshape(x.shape[0], h * w, c)


@jax.jit
def kernel(feat1, feat2, feat3, mem1, mem2, mem3):
    fn1 = _nhwc(feat1, *_SHAPES[0])
    mn1 = _nhwc(mem1, *_SHAPES[0])
    fn2 = _nhwc(feat2, *_SHAPES[1])
    mn2 = _nhwc(mem2, *_SHAPES[1])
    fn3 = _nhwc(feat3, *_SHAPES[2])
    mn3 = _nhwc(mem3, *_SHAPES[2])

    ff1 = feat1.reshape(_B, -1)
    mf1 = mem1.reshape(_M, -1)
    ff2 = fn2.reshape(_B, -1)
    mf2 = mn2.reshape(_M, -1)
    ff3 = fn3.reshape(_B, -1)
    mf3 = mn3.reshape(_M, -1)

    idx = _compute_idx(ff1, mf1, ff2, mf2, ff3, mf3)
    return tuple(_compute_outputs(idx, fn1, mn1, fn2, mn2, fn3, mn3))
